# baseline (speedup 1.0000x reference)
import functools

import jax
import jax.numpy as jnp
from jax import lax
from jax.experimental import pallas as pl
from jax.experimental.pallas import tpu as pltpu
from jax.experimental.pallas import tpu_sc as plsc

NC = 2
NS = 16
NW = NC * NS
L = 16

BATCH = 4096
HIST = 200
HIDDEN = 64
N_NODE = 1000000
NBLK = (N_NODE + 127) // 128
N_PAD = NBLK * 128
BH = BATCH // 128
UNITS = (HIST // 8) * BH


def _rsqrt_vec(s):
    i = lax.bitcast_convert_type(s, jnp.int32)
    i = jnp.int32(0x5F3759DF) - (i >> 1)
    y = lax.bitcast_convert_type(i, jnp.float32)
    for _ in range(2):
        y = y * (1.5 - 0.5 * s * y * y)
    return y


@functools.partial(
    pl.kernel,
    out_type=jax.ShapeDtypeStruct((N_PAD, 128), jnp.float32),
    mesh=plsc.VectorSubcoreMesh(
        core_axis_name="c", subcore_axis_name="s", num_cores=NC
    ),
    compiler_params=pltpu.CompilerParams(use_tc_tiling_on_sc=True, needs_layout_passes=False),
    scratch_types=[
        pltpu.VMEM((2, HIDDEN, 128), jnp.float32),
        pltpu.VMEM((2, 128, 128), jnp.float32),
        pltpu.SemaphoreType.DMA((2,)),
        pltpu.SemaphoreType.DMA((2,)),
    ],
)
def _table_repack(tt_hbm, out_hbm, tin, tout, isem, osem):
    wid = lax.axis_index("s") * NC + lax.axis_index("c")
    lanes = lax.iota(jnp.int32, L)

    def _in_copy(c, b):
        col = pl.multiple_of(c * 128, 128)
        return pltpu.make_async_copy(
            tt_hbm.at[:, pl.ds(col, 128)], tin.at[b], isem.at[b]
        )

    def _out_copy(c, b):
        row = pl.multiple_of(c * 128, 128)
        return pltpu.make_async_copy(
            tout.at[b], out_hbm.at[pl.ds(row, 128)], osem.at[b]
        )

    def _transpose(b):
        for nq in range(8):
            idx_n = lanes + (nq * L)
            for f in range(HIDDEN):
                v = tin[b, f, pl.ds(nq * L, L)]
                plsc.store_scatter(
                    tout.at[b], [idx_n, jnp.full((L,), f, jnp.int32)], v
                )

    nk = (NBLK + 2 * NW - 1) // (2 * NW)

    def block(k, carry):
        for b in range(2):
            c = jnp.minimum((2 * k + b) * NW + wid, NBLK - 1)
            _in_copy(c, b).start()
            _in_copy(c, b).wait()

            @pl.when(k > 0)
            def _():
                _out_copy(0, b).wait()

            _transpose(b)
            _out_copy(c, b).start()
        return carry

    lax.fori_loop(0, nk, block, 0)
    for b in range(2):
        _out_copy(0, b).wait()


@functools.partial(
    pl.kernel,
    out_type=jax.ShapeDtypeStruct((HIST, 8, BH, 8, 128), jnp.float32),
    mesh=plsc.VectorSubcoreMesh(
        core_axis_name="c", subcore_axis_name="s", num_cores=NC
    ),
    compiler_params=pltpu.CompilerParams(use_tc_tiling_on_sc=True, needs_layout_passes=False),
    scratch_types=[
        pltpu.VMEM((8, 128), jnp.int32),
        pltpu.VMEM((2, 128, 128), jnp.float32),
        pltpu.VMEM((2, 8, 8, 128), jnp.float32),
        pltpu.SemaphoreType.DMA,
        pltpu.SemaphoreType.DMA((2,)),
        pltpu.SemaphoreType.DMA((2,)),
    ],
)
def _gather_norm(idxt_hbm, rows_hbm, out_hbm, idxv, gbuf, obuf, isem, gsem, osem):
    wid = lax.axis_index("s") * NC + lax.axis_index("c")
    lanes = lax.iota(jnp.int32, L)

    def _gather(hh, b):
        return pltpu.make_async_copy(
            rows_hbm.at[idxv.at[hh]], gbuf.at[b], gsem.at[b]
        )

    def _put(h, bh, b):
        return pltpu.make_async_copy(
            obuf.at[b], out_hbm.at[h, :, bh], osem.at[b]
        )

    def _normalize(b):
        for q in range(8):
            idx_b = lanes + (q * L)
            ss = jnp.zeros((L,), jnp.float32)
            for f in range(HIDDEN):
                v = plsc.load_gather(
                    gbuf.at[b], [idx_b, jnp.full((L,), f, jnp.int32)]
                )
                ss = ss + v * v
            sc = _rsqrt_vec(ss)
            for f in range(HIDDEN):
                v = plsc.load_gather(
                    gbuf.at[b], [idx_b, jnp.full((L,), f, jnp.int32)]
                )
                obuf[b, f // 8, f % 8, pl.ds(q * L, L)] = v * sc

    def unit(k, carry):
        u = k * NW + wid
        h8 = u // BH
        bh = u % BH
        pltpu.make_async_copy(
            idxt_hbm.at[pl.ds(pl.multiple_of(h8 * 8, 8), 8),
                        pl.ds(pl.multiple_of(bh * 128, 128), 128)],
            idxv,
            isem,
        ).start()
        pltpu.make_async_copy(
            idxt_hbm.at[pl.ds(0, 8), pl.ds(0, 128)], idxv, isem
        ).wait()
        _gather(0, 0).start()

        def pair(p, c2):
            for b in range(2):
                hh = p * 2 + b

                @pl.when(hh + 1 < 8)
                def _():
                    _gather(hh + 1, 1 - b).start()

                _gather(hh, b).wait()

                @pl.when((k > 0) | (p > 0))
                def _():
                    _put(0, 0, b).wait()

                _normalize(b)
                _put(h8 * 8 + hh, bh, b).start()
            return c2

        lax.fori_loop(0, 4, pair, 0)
        return carry

    lax.fori_loop(0, UNITS // NW, unit, 0)
    for b in range(2):
        _put(0, 0, b).wait()


def kernel(inputs, embedding_weight):
    idxt = inputs.T.astype(jnp.int32)
    tablet = embedding_weight.T
    rows = _table_repack(tablet)
    out5 = _gather_norm(idxt, rows)
    return (
        out5.transpose(2, 4, 0, 1, 3)
        .reshape(BATCH, HIST, HIDDEN)
    )


# speedup vs baseline: 1.0800x; 1.0800x over previous
import functools

import jax
import jax.numpy as jnp
from jax import lax
from jax.experimental import pallas as pl
from jax.experimental.pallas import tpu as pltpu
from jax.experimental.pallas import tpu_sc as plsc

NC = 2
NS = 16
NW = NC * NS
L = 16

BATCH = 4096
HIST = 200
HIDDEN = 64
N_NODE = 1000000
NBLK = (N_NODE + 127) // 128
N_PAD = NBLK * 128
BH = BATCH // 128
UNITS = (HIST // 8) * BH


def _rsqrt_vec(s):
    i = lax.bitcast_convert_type(s, jnp.int32)
    i = jnp.int32(0x5F3759DF) - (i >> 1)
    y = lax.bitcast_convert_type(i, jnp.float32)
    for _ in range(2):
        y = y * (1.5 - 0.5 * s * y * y)
    return y


@functools.partial(
    pl.kernel,
    out_type=jax.ShapeDtypeStruct((N_PAD, 128), jnp.float32),
    mesh=plsc.VectorSubcoreMesh(
        core_axis_name="c", subcore_axis_name="s", num_cores=NC
    ),
    compiler_params=pltpu.CompilerParams(use_tc_tiling_on_sc=True, needs_layout_passes=False),
    scratch_types=[
        pltpu.VMEM((2, HIDDEN, 128), jnp.float32),
        pltpu.VMEM((2, 128, 128), jnp.float32),
        pltpu.SemaphoreType.DMA((2,)),
        pltpu.SemaphoreType.DMA((2,)),
    ],
)
def _table_repack(tt_hbm, out_hbm, tin, tout, isem, osem):
    wid = lax.axis_index("s") * NC + lax.axis_index("c")
    lanes = lax.iota(jnp.int32, L)

    def _in_copy(c, b):
        col = pl.multiple_of(c * 128, 128)
        return pltpu.make_async_copy(
            tt_hbm.at[:, pl.ds(col, 128)], tin.at[b], isem.at[b]
        )

    def _out_copy(c, b):
        row = pl.multiple_of(c * 128, 128)
        return pltpu.make_async_copy(
            tout.at[b], out_hbm.at[pl.ds(row, 128)], osem.at[b]
        )

    def _transpose(b):
        for nq in range(8):
            idx_n = lanes + (nq * L)
            for f in range(HIDDEN):
                v = tin[b, f, pl.ds(nq * L, L)]
                plsc.store_scatter(
                    tout.at[b], [idx_n, jnp.full((L,), f, jnp.int32)], v
                )

    nk = (NBLK + 2 * NW - 1) // (2 * NW)

    def _blk(k, b):
        return jnp.minimum((2 * k + b) * NW + wid, NBLK - 1)

    for b in range(2):
        _in_copy(_blk(0, b), b).start()

    def block(k, carry):
        for b in range(2):
            c = _blk(k, b)
            _in_copy(c, b).wait()

            @pl.when(k > 0)
            def _():
                _out_copy(0, b).wait()

            _transpose(b)
            _out_copy(c, b).start()

            @pl.when(k < nk - 1)
            def _():
                _in_copy(_blk(k + 1, b), b).start()
        return carry

    lax.fori_loop(0, nk, block, 0)
    for b in range(2):
        _out_copy(0, b).wait()


@functools.partial(
    pl.kernel,
    out_type=jax.ShapeDtypeStruct((HIST, 8, BH, 8, 128), jnp.float32),
    mesh=plsc.VectorSubcoreMesh(
        core_axis_name="c", subcore_axis_name="s", num_cores=NC
    ),
    compiler_params=pltpu.CompilerParams(use_tc_tiling_on_sc=True, needs_layout_passes=False),
    scratch_types=[
        pltpu.VMEM((2, 8, 128), jnp.int32),
        pltpu.VMEM((2, 128, 128), jnp.float32),
        pltpu.VMEM((2, 8, 8, 128), jnp.float32),
        pltpu.SemaphoreType.DMA((2,)),
        pltpu.SemaphoreType.DMA((2,)),
        pltpu.SemaphoreType.DMA((2,)),
    ],
)
def _gather_norm(idxt_hbm, rows_hbm, out_hbm, idxv, gbuf, obuf, isem, gsem, osem):
    wid = lax.axis_index("s") * NC + lax.axis_index("c")
    lanes = lax.iota(jnp.int32, L)
    nu2 = (UNITS // NW + 1) // 2

    def _u(k2, ib):
        return jnp.minimum(k2 * 2 + ib, UNITS // NW - 1) * NW + wid

    def _icopy(u, ib):
        h8 = u // BH
        bh = u % BH
        return pltpu.make_async_copy(
            idxt_hbm.at[pl.ds(pl.multiple_of(h8 * 8, 8), 8),
                        pl.ds(pl.multiple_of(bh * 128, 128), 128)],
            idxv.at[ib],
            isem.at[ib],
        )

    def _gather(ib, hh, b):
        return pltpu.make_async_copy(
            rows_hbm.at[idxv.at[ib, hh]], gbuf.at[b], gsem.at[b]
        )

    def _put(h, bh, b):
        return pltpu.make_async_copy(
            obuf.at[b], out_hbm.at[h, :, bh], osem.at[b]
        )

    def _normalize(b):
        def qbody(q, cq):
            idx_b = lanes + q * L
            ss = jnp.zeros((L,), jnp.float32)
            for f in range(HIDDEN):
                v = plsc.load_gather(
                    gbuf.at[b], [idx_b, jnp.full((L,), f, jnp.int32)]
                )
                ss = ss + v * v
            sc = _rsqrt_vec(ss)
            q16 = q * L
            for f in range(HIDDEN):
                v = plsc.load_gather(
                    gbuf.at[b], [idx_b, jnp.full((L,), f, jnp.int32)]
                )
                obuf[b, f // 8, f % 8, pl.ds(q16, L)] = v * sc
            return cq

        lax.fori_loop(0, 8, qbody, 0)

    _icopy(_u(0, 0), 0).start()

    def unit2(k2, carry):
        for ib in range(2):
            u = _u(k2, ib)
            h8 = u // BH
            bh = u % BH
            nxt = _u(k2 + (1 if ib == 1 else 0), 1 - ib)
            _icopy(nxt, 1 - ib).start()
            _icopy(u, ib).wait()
            _gather(ib, 0, 0).start()

            def pair(p, c2):
                for b in range(2):
                    hh = p * 2 + b

                    @pl.when(hh + 1 < 8)
                    def _():
                        _gather(ib, hh + 1, 1 - b).start()

                    _gather(ib, hh, b).wait()

                    @pl.when((k2 > 0) | (ib > 0) | (p > 0))
                    def _():
                        _put(0, 0, b).wait()

                    _normalize(b)
                    _put(h8 * 8 + hh, bh, b).start()
                return c2

            lax.fori_loop(0, 4, pair, 0)
        return carry

    lax.fori_loop(0, nu2, unit2, 0)
    _icopy(_u(0, 0), 0).wait()
    for b in range(2):
        _put(0, 0, b).wait()


def kernel(inputs, embedding_weight):
    idxt = inputs.T.astype(jnp.int32)
    tablet = embedding_weight.T
    rows = _table_repack(tablet)
    out5 = _gather_norm(idxt, rows)
    return (
        out5.transpose(2, 4, 0, 1, 3)
        .reshape(BATCH, HIST, HIDDEN)
    )


# speedup vs baseline: 1.0810x; 1.0010x over previous
import functools

import jax
import jax.numpy as jnp
from jax import lax
from jax.experimental import pallas as pl
from jax.experimental.pallas import tpu as pltpu
from jax.experimental.pallas import tpu_sc as plsc

NC = 2
NS = 16
NW = NC * NS
L = 16

BATCH = 4096
HIST = 200
HIDDEN = 64
N_NODE = 1000000
NBLK = (N_NODE + 127) // 128
N_PAD = NBLK * 128
BH = BATCH // 128
UNITS = (HIST // 8) * BH


def _rsqrt_vec(s):
    i = lax.bitcast_convert_type(s, jnp.int32)
    i = jnp.int32(0x5F3759DF) - (i >> 1)
    y = lax.bitcast_convert_type(i, jnp.float32)
    for _ in range(2):
        y = y * (1.5 - 0.5 * s * y * y)
    return y


@functools.partial(
    pl.kernel,
    out_type=jax.ShapeDtypeStruct((N_PAD, 128), jnp.float32),
    mesh=plsc.VectorSubcoreMesh(
        core_axis_name="c", subcore_axis_name="s", num_cores=NC
    ),
    compiler_params=pltpu.CompilerParams(use_tc_tiling_on_sc=True, needs_layout_passes=False),
    scratch_types=[
        pltpu.VMEM((2, HIDDEN, 128), jnp.float32),
        pltpu.VMEM((2, 128, 128), jnp.float32),
        pltpu.SemaphoreType.DMA((2,)),
        pltpu.SemaphoreType.DMA((2,)),
    ],
)
def _table_repack(tt_hbm, out_hbm, tin, tout, isem, osem):
    wid = lax.axis_index("s") * NC + lax.axis_index("c")
    lanes = lax.iota(jnp.int32, L)

    def _in_copy(c, b):
        col = pl.multiple_of(c * 128, 128)
        return pltpu.make_async_copy(
            tt_hbm.at[:, pl.ds(col, 128)], tin.at[b], isem.at[b]
        )

    def _out_copy(c, b):
        row = pl.multiple_of(c * 128, 128)
        return pltpu.make_async_copy(
            tout.at[b], out_hbm.at[pl.ds(row, 128)], osem.at[b]
        )

    def _transpose(b):
        for nq in range(8):
            idx_n = lanes + (nq * L)
            for f in range(HIDDEN):
                v = tin[b, f, pl.ds(nq * L, L)]
                plsc.store_scatter(
                    tout.at[b], [idx_n, jnp.full((L,), f, jnp.int32)], v
                )

    nk = (NBLK + 2 * NW - 1) // (2 * NW)

    def _blk(k, b):
        return jnp.minimum((2 * k + b) * NW + wid, NBLK - 1)

    for b in range(2):
        _in_copy(_blk(0, b), b).start()

    def block(k, carry):
        for b in range(2):
            c = _blk(k, b)
            _in_copy(c, b).wait()

            @pl.when(k > 0)
            def _():
                _out_copy(0, b).wait()

            _transpose(b)
            _out_copy(c, b).start()

            @pl.when(k < nk - 1)
            def _():
                _in_copy(_blk(k + 1, b), b).start()
        return carry

    lax.fori_loop(0, nk, block, 0)
    for b in range(2):
        _out_copy(0, b).wait()


@functools.partial(
    pl.kernel,
    out_type=jax.ShapeDtypeStruct((HIST, 8, BH, 8, 128), jnp.float32),
    mesh=plsc.VectorSubcoreMesh(
        core_axis_name="c", subcore_axis_name="s", num_cores=NC
    ),
    compiler_params=pltpu.CompilerParams(needs_layout_passes=False),
    scratch_types=[
        pltpu.VMEM((2, 8, 128), jnp.int32),
        pltpu.VMEM((2, 128, 128), jnp.float32),
        pltpu.VMEM((2, 8, 8, 128), jnp.float32),
        pltpu.SemaphoreType.DMA((2,)),
        pltpu.SemaphoreType.DMA((2,)),
        pltpu.SemaphoreType.DMA((2,)),
    ],
)
def _gather_norm(idxt_hbm, rows_hbm, out_hbm, idxv, gbuf, obuf, isem, gsem, osem):
    wid = lax.axis_index("s") * NC + lax.axis_index("c")
    lanes = lax.iota(jnp.int32, L)
    nu2 = (UNITS // NW + 1) // 2

    def _u(k2, ib):
        return jnp.minimum(k2 * 2 + ib, UNITS // NW - 1) * NW + wid

    def _icopy(u, ib):
        h8 = u // BH
        bh = u % BH
        return pltpu.make_async_copy(
            idxt_hbm.at[pl.ds(pl.multiple_of(h8 * 8, 8), 8),
                        pl.ds(pl.multiple_of(bh * 128, 128), 128)],
            idxv.at[ib],
            isem.at[ib],
        )

    def _gather(ib, hh, b):
        return pltpu.make_async_copy(
            rows_hbm.at[idxv.at[ib, hh]], gbuf.at[b], gsem.at[b]
        )

    def _put(h, bh, b):
        return pltpu.make_async_copy(
            obuf.at[b], out_hbm.at[h, :, bh], osem.at[b]
        )

    def _normalize(b):
        def qbody(q, cq):
            idx_b = lanes + q * L
            ss = jnp.zeros((L,), jnp.float32)
            for f in range(HIDDEN):
                v = plsc.load_gather(
                    gbuf.at[b], [idx_b, jnp.full((L,), f, jnp.int32)]
                )
                ss = ss + v * v
            sc = _rsqrt_vec(ss)
            q16 = q * L
            for f in range(HIDDEN):
                v = plsc.load_gather(
                    gbuf.at[b], [idx_b, jnp.full((L,), f, jnp.int32)]
                )
                obuf[b, f // 8, f % 8, pl.ds(q16, L)] = v * sc
            return cq

        lax.fori_loop(0, 8, qbody, 0)

    _icopy(_u(0, 0), 0).start()

    def unit2(k2, carry):
        for ib in range(2):
            u = _u(k2, ib)
            h8 = u // BH
            bh = u % BH
            nxt = _u(k2 + (1 if ib == 1 else 0), 1 - ib)
            _icopy(nxt, 1 - ib).start()
            _icopy(u, ib).wait()
            _gather(ib, 0, 0).start()

            def pair(p, c2):
                for b in range(2):
                    hh = p * 2 + b

                    @pl.when(hh + 1 < 8)
                    def _():
                        _gather(ib, hh + 1, 1 - b).start()

                    _gather(ib, hh, b).wait()

                    @pl.when((k2 > 0) | (ib > 0) | (p > 0))
                    def _():
                        _put(0, 0, b).wait()

                    _normalize(b)
                    _put(h8 * 8 + hh, bh, b).start()
                return c2

            lax.fori_loop(0, 4, pair, 0)
        return carry

    lax.fori_loop(0, nu2, unit2, 0)
    _icopy(_u(0, 0), 0).wait()
    for b in range(2):
        _put(0, 0, b).wait()


def kernel(inputs, embedding_weight):
    idxt = inputs.T.astype(jnp.int32)
    tablet = embedding_weight.T
    rows = _table_repack(tablet)
    out5 = _gather_norm(idxt, rows)
    return (
        out5.transpose(2, 4, 0, 1, 3)
        .reshape(BATCH, HIST, HIDDEN)
    )


# speedup vs baseline: 1.6553x; 1.5312x over previous
import functools

import jax
import jax.numpy as jnp
from jax import lax
from jax.experimental import pallas as pl
from jax.experimental.pallas import tpu as pltpu
from jax.experimental.pallas import tpu_sc as plsc

NC = 2
NS = 16
NW = NC * NS
L = 16

BATCH = 4096
HIST = 200
HIDDEN = 64
N_NODE = 1000000
NBLK = (N_NODE + 127) // 128
N_PAD = NBLK * 128
BH = BATCH // 128
UNITS = (HIST // 8) * BH


def _rsqrt_vec(s):
    i = lax.bitcast_convert_type(s, jnp.int32)
    i = jnp.int32(0x5F3759DF) - (i >> 1)
    y = lax.bitcast_convert_type(i, jnp.float32)
    for _ in range(2):
        y = y * (1.5 - 0.5 * s * y * y)
    return y


@functools.partial(
    pl.kernel,
    out_type=jax.ShapeDtypeStruct((N_PAD, 128), jnp.float32),
    mesh=plsc.VectorSubcoreMesh(
        core_axis_name="c", subcore_axis_name="s", num_cores=NC
    ),
    compiler_params=pltpu.CompilerParams(use_tc_tiling_on_sc=True, needs_layout_passes=False),
    scratch_types=[
        pltpu.VMEM((2, HIDDEN, 128), jnp.float32),
        pltpu.VMEM((2, 128, 128), jnp.float32),
        pltpu.SemaphoreType.DMA((2,)),
        pltpu.SemaphoreType.DMA((2,)),
    ],
)
def _table_repack(tt_hbm, out_hbm, tin, tout, isem, osem):
    wid = lax.axis_index("s") * NC + lax.axis_index("c")
    lanes = lax.iota(jnp.int32, L)

    def _in_copy(c, b):
        col = pl.multiple_of(c * 128, 128)
        return pltpu.make_async_copy(
            tt_hbm.at[:, pl.ds(col, 128)], tin.at[b], isem.at[b]
        )

    def _out_copy(c, b):
        row = pl.multiple_of(c * 128, 128)
        return pltpu.make_async_copy(
            tout.at[b], out_hbm.at[pl.ds(row, 128)], osem.at[b]
        )

    fzero = jnp.zeros((L,), jnp.int32)
    fq_idx = [lanes + fq * L for fq in range(4)]

    def _transpose(b):
        for nq in range(8):
            for j in range(L):
                n = nq * L + j
                idx_nv = fzero + n
                vs = [
                    plsc.load_gather(tin.at[b], [fq_idx[fq], idx_nv])
                    for fq in range(4)
                ]
                for fq in range(4):
                    tout[b, n, pl.ds(fq * L, L)] = vs[fq]

    nk = (NBLK + 2 * NW - 1) // (2 * NW)

    def _blk(k, b):
        return jnp.minimum((2 * k + b) * NW + wid, NBLK - 1)

    for b in range(2):
        _in_copy(_blk(0, b), b).start()

    def block(k, carry):
        for b in range(2):
            c = _blk(k, b)
            _in_copy(c, b).wait()

            @pl.when(k > 0)
            def _():
                _out_copy(0, b).wait()

            _transpose(b)
            _out_copy(c, b).start()

            @pl.when(k < nk - 1)
            def _():
                _in_copy(_blk(k + 1, b), b).start()
        return carry

    lax.fori_loop(0, nk, block, 0)
    for b in range(2):
        _out_copy(0, b).wait()


@functools.partial(
    pl.kernel,
    out_type=jax.ShapeDtypeStruct((HIST, 8, BH, 8, 128), jnp.float32),
    mesh=plsc.VectorSubcoreMesh(
        core_axis_name="c", subcore_axis_name="s", num_cores=NC
    ),
    compiler_params=pltpu.CompilerParams(needs_layout_passes=False),
    scratch_types=[
        pltpu.VMEM((2, 8, 128), jnp.int32),
        pltpu.VMEM((2, 128, 128), jnp.float32),
        pltpu.VMEM((2, 8, 8, 128), jnp.float32),
        pltpu.SemaphoreType.DMA((2,)),
        pltpu.SemaphoreType.DMA((2,)),
        pltpu.SemaphoreType.DMA((2,)),
    ],
)
def _gather_norm(idxt_hbm, rows_hbm, out_hbm, idxv, gbuf, obuf, isem, gsem, osem):
    wid = lax.axis_index("s") * NC + lax.axis_index("c")
    lanes = lax.iota(jnp.int32, L)
    nu2 = (UNITS // NW + 1) // 2

    def _u(k2, ib):
        return jnp.minimum(k2 * 2 + ib, UNITS // NW - 1) * NW + wid

    def _icopy(u, ib):
        h8 = u // BH
        bh = u % BH
        return pltpu.make_async_copy(
            idxt_hbm.at[pl.ds(pl.multiple_of(h8 * 8, 8), 8),
                        pl.ds(pl.multiple_of(bh * 128, 128), 128)],
            idxv.at[ib],
            isem.at[ib],
        )

    def _gather(ib, hh, b):
        return pltpu.make_async_copy(
            rows_hbm.at[idxv.at[ib, hh]], gbuf.at[b], gsem.at[b]
        )

    def _put(h, bh, b):
        return pltpu.make_async_copy(
            obuf.at[b], out_hbm.at[h, :, bh], osem.at[b]
        )

    fzero = jnp.zeros((L,), jnp.int32)

    def _normalize(b):
        def qbody(q, cq):
            idx_b = lanes + q * L
            q16 = q * L
            acc = [jnp.zeros((L,), jnp.float32) for _ in range(4)]
            for f0 in range(0, HIDDEN, 8):
                vs = [
                    plsc.load_gather(gbuf.at[b], [idx_b, fzero + (f0 + j)])
                    for j in range(8)
                ]
                for j in range(8):
                    f = f0 + j
                    obuf[b, f // 8, f % 8, pl.ds(q16, L)] = vs[j]
                    acc[j % 4] = acc[j % 4] + vs[j] * vs[j]
            sc = _rsqrt_vec((acc[0] + acc[1]) + (acc[2] + acc[3]))
            for f0 in range(0, HIDDEN, 8):
                ws = [
                    obuf[b, (f0 + j) // 8, (f0 + j) % 8, pl.ds(q16, L)]
                    for j in range(8)
                ]
                for j in range(8):
                    f = f0 + j
                    obuf[b, f // 8, f % 8, pl.ds(q16, L)] = ws[j] * sc
            return cq

        lax.fori_loop(0, 8, qbody, 0)

    _icopy(_u(0, 0), 0).start()

    def unit2(k2, carry):
        for ib in range(2):
            u = _u(k2, ib)
            h8 = u // BH
            bh = u % BH
            nxt = _u(k2 + (1 if ib == 1 else 0), 1 - ib)
            _icopy(nxt, 1 - ib).start()
            _icopy(u, ib).wait()
            _gather(ib, 0, 0).start()

            def pair(p, c2):
                for b in range(2):
                    hh = p * 2 + b

                    @pl.when(hh + 1 < 8)
                    def _():
                        _gather(ib, hh + 1, 1 - b).start()

                    _gather(ib, hh, b).wait()

                    @pl.when((k2 > 0) | (ib > 0) | (p > 0))
                    def _():
                        _put(0, 0, b).wait()

                    _normalize(b)
                    _put(h8 * 8 + hh, bh, b).start()
                return c2

            lax.fori_loop(0, 4, pair, 0)
        return carry

    lax.fori_loop(0, nu2, unit2, 0)
    _icopy(_u(0, 0), 0).wait()
    for b in range(2):
        _put(0, 0, b).wait()


def kernel(inputs, embedding_weight):
    idxt = inputs.T.astype(jnp.int32)
    tablet = embedding_weight.T
    rows = _table_repack(tablet)
    out5 = _gather_norm(idxt, rows)
    return (
        out5.transpose(2, 4, 0, 1, 3)
        .reshape(BATCH, HIST, HIDDEN)
    )
